# baseline (speedup 1.0000x reference)
import functools

import jax
import jax.numpy as jnp
from jax import lax
from jax.experimental import pallas as pl
from jax.experimental.pallas import tpu as pltpu
from jax.experimental.pallas import tpu_sc as plsc

NC = 2
NS = 16
NW = NC * NS
D = 128
G = 100
GP = 104
EK = 64
NB = 4


@functools.cache
def _mesh():
    return plsc.VectorSubcoreMesh(core_axis_name="c", subcore_axis_name="s",
                                  num_cores=NC, num_subcores=NS)


def _prep_body(npad, ngrp, x_hbm, dst_hbm, emb_hbm, z128_hbm, ones_hbm,
               deg_hbm, h0_hbm, idxv, rows_v, ones_v, acc, isem, gsem, ssem):
    c = lax.axis_index("c")
    s = lax.axis_index("s")
    w = c * NS + s
    rpw = npad // NW
    rps = npad // NS

    nch = rpw // EK
    for start in range(0, nch, NB):
        bs = range(min(NB, nch - start))
        dl = [pltpu.async_copy(
            x_hbm.at[pl.ds(w * rpw + (start + b) * EK, EK)],
            idxv.at[b], isem) for b in bs]
        for dd in dl:
            dd.wait()
        dl = [pltpu.async_copy(emb_hbm.at[idxv.at[b]], rows_v.at[b], gsem)
              for b in bs]
        for dd in dl:
            dd.wait()
        dl = [pltpu.async_copy(
            rows_v.at[b], h0_hbm.at[pl.ds(w * rpw + (start + b) * EK, EK)],
            ssem) for b in bs]
        for dd in dl:
            dd.wait()

    pltpu.sync_copy(z128_hbm, acc.at[pl.ds(s * rps, rps)])
    pltpu.sync_copy(ones_hbm, ones_v)
    plsc.subcore_barrier()

    epw = ngrp * NB * EK

    def grp(gi, _):
        ds1 = [pltpu.async_copy(
            dst_hbm.at[pl.ds(w * epw + (gi * NB + b) * EK, EK)],
            idxv.at[b], isem) for b in range(NB)]
        for dd in ds1:
            dd.wait()
        ds2 = [pltpu.async_copy(ones_v, acc.at[idxv.at[b]], ssem, add=True)
               for b in range(NB)]
        for dd in ds2:
            dd.wait()
        return 0

    lax.fori_loop(0, ngrp, grp, 0)
    plsc.subcore_barrier()
    pltpu.sync_copy(acc.at[pl.ds(s * rps, rps)],
                    deg_hbm.at[c, pl.ds(s * rps, rps)])


def _scatter_body(npad, ngrp, g_hbm, src_hbm, dst_hbm, z128_hbm,
                  sh_hbm, srcv, dstv, rows_v, acc, isem, gsem, ssem):
    c = lax.axis_index("c")
    s = lax.axis_index("s")
    w = c * NS + s
    rps = npad // NS

    pltpu.sync_copy(z128_hbm, acc.at[pl.ds(s * rps, rps)])
    plsc.subcore_barrier()

    epw = ngrp * NB * EK

    def grp(gi, _):
        ds1 = []
        for b in range(NB):
            base = w * epw + (gi * NB + b) * EK
            ds1.append(pltpu.async_copy(src_hbm.at[pl.ds(base, EK)],
                                        srcv.at[b], isem))
            ds1.append(pltpu.async_copy(dst_hbm.at[pl.ds(base, EK)],
                                        dstv.at[b], isem))
        for dd in ds1:
            dd.wait()
        ds2 = [pltpu.async_copy(g_hbm.at[srcv.at[b]], rows_v.at[b], gsem)
               for b in range(NB)]
        for dd in ds2:
            dd.wait()
        ds3 = [pltpu.async_copy(rows_v.at[b], acc.at[dstv.at[b]], ssem,
                                add=True) for b in range(NB)]
        for dd in ds3:
            dd.wait()
        return 0

    lax.fori_loop(0, ngrp, grp, 0)
    plsc.subcore_barrier()
    pltpu.sync_copy(acc.at[pl.ds(s * rps, rps)],
                    sh_hbm.at[c, pl.ds(s * rps, rps)])


def _pool_body(npad, h3_hbm, batch_hbm, zpool_hbm, pool_hbm,
               buf, rows_v, bv):
    c = lax.axis_index("c")
    s = lax.axis_index("s")
    w = c * NS + s
    rpw = npad // NW

    pltpu.sync_copy(zpool_hbm, buf)
    base = w * rpw
    pltpu.sync_copy(h3_hbm.at[pl.ds(base * D, rpw * D)], rows_v)
    pltpu.sync_copy(batch_hbm.at[pl.ds(base, rpw)], bv)

    def row_blk(jb, _):
        bvec = bv[pl.ds(jb * 16, 16)]
        for ii in range(16):
            gb = bvec[ii] * D
            rb = (jb * 16 + ii) * D
            for k in range(D // 16):
                v = rows_v[pl.ds(rb + k * 16, 16)]
                cur = buf[pl.ds(gb + k * 16, 16)]
                buf[pl.ds(gb + k * 16, 16)] = jnp.maximum(cur, v)
        return 0

    lax.fori_loop(0, rpw // 16, row_blk, 0)
    pltpu.sync_copy(buf, pool_hbm.at[w])


@functools.cache
def _make_prep(npad, ngrp):
    return pl.kernel(
        functools.partial(_prep_body, npad, ngrp),
        out_type=(jax.ShapeDtypeStruct((NC, npad, D), jnp.float32),
                  jax.ShapeDtypeStruct((npad, D), jnp.float32)),
        mesh=_mesh(),
        scratch_types=[
            pltpu.VMEM((NB, EK), jnp.int32),
            pltpu.VMEM((NB, EK, D), jnp.float32),
            pltpu.VMEM((EK, D), jnp.float32),
            pltpu.VMEM_SHARED((npad, D), jnp.float32),
            pltpu.SemaphoreType.DMA,
            pltpu.SemaphoreType.DMA,
            pltpu.SemaphoreType.DMA,
        ],
    )


@functools.cache
def _make_scatter(npad, ngrp):
    return pl.kernel(
        functools.partial(_scatter_body, npad, ngrp),
        out_type=jax.ShapeDtypeStruct((NC, npad, D), jnp.float32),
        mesh=_mesh(),
        scratch_types=[
            pltpu.VMEM((NB, EK), jnp.int32),
            pltpu.VMEM((NB, EK), jnp.int32),
            pltpu.VMEM((NB, EK, D), jnp.float32),
            pltpu.VMEM_SHARED((npad, D), jnp.float32),
            pltpu.SemaphoreType.DMA,
            pltpu.SemaphoreType.DMA,
            pltpu.SemaphoreType.DMA,
        ],
    )


@functools.cache
def _make_pool(npad):
    rpw = npad // NW
    return pl.kernel(
        functools.partial(_pool_body, npad),
        out_type=jax.ShapeDtypeStruct((NW, GP * D), jnp.float32),
        mesh=_mesh(),
        scratch_types=[
            pltpu.VMEM((GP * D,), jnp.float32),
            pltpu.VMEM((rpw * D,), jnp.float32),
            pltpu.VMEM((rpw,), jnp.int32),
        ],
    )


BLK = 256


def _rowmask(nreal, val):
    ridx = pl.program_id(0) * BLK + lax.broadcasted_iota(jnp.int32, val.shape, 0)
    return jnp.where(ridx < nreal, val, 0.0)


def _tc1_body(nreal, dega, degb, h0, W, dinv_o, g_o):
    dinv = lax.rsqrt(dega[:, :1] + degb[:, :1] + 1.0)
    dinv_o[...] = dinv
    g = dinv * jnp.dot(h0[...], W[...], preferred_element_type=jnp.float32)
    g_o[...] = _rowmask(nreal, g)


def _tc_mid_body(nreal, sa, sb, g, dinv, b, W, out):
    dv = dinv[...]
    h = jnp.maximum(dv * (sa[...] + sb[...] + g[...]) + b[...], 0.0)
    out[...] = _rowmask(nreal, dv * jnp.dot(h, W[...],
                                            preferred_element_type=jnp.float32))


def _tc_last_body(nreal, sa, sb, g, dinv, b, out):
    h = jnp.maximum(dinv[...] * (sa[...] + sb[...] + g[...]) + b[...], 0.0)
    out[...] = _rowmask(nreal, h)


def _tc_pool_body(pool, Wf, bf, out):
    pooled = jnp.max(pool[...], axis=0)
    out[...] = jnp.dot(pooled, Wf[...], preferred_element_type=jnp.float32) + bf[0, 0]


def _row_spec(width):
    return pl.BlockSpec((BLK, width), lambda i: (i, 0))


def _full_spec(shape):
    return pl.BlockSpec(shape, lambda i: tuple(0 for _ in shape))


@functools.cache
def _make_tc1(npad, nreal):
    return pl.pallas_call(
        functools.partial(_tc1_body, nreal),
        grid=(npad // BLK,),
        in_specs=[_row_spec(D), _row_spec(D), _row_spec(D), _full_spec((D, D))],
        out_specs=[_row_spec(1), _row_spec(D)],
        out_shape=(jax.ShapeDtypeStruct((npad, 1), jnp.float32),
                   jax.ShapeDtypeStruct((npad, D), jnp.float32)),
    )


@functools.cache
def _make_tc_mid(npad, nreal):
    return pl.pallas_call(
        functools.partial(_tc_mid_body, nreal),
        grid=(npad // BLK,),
        in_specs=[_row_spec(D), _row_spec(D), _row_spec(D), _row_spec(1),
                  _full_spec((1, D)), _full_spec((D, D))],
        out_specs=_row_spec(D),
        out_shape=jax.ShapeDtypeStruct((npad, D), jnp.float32),
    )


@functools.cache
def _make_tc_last(npad, nreal):
    return pl.pallas_call(
        functools.partial(_tc_last_body, nreal),
        grid=(npad // BLK,),
        in_specs=[_row_spec(D), _row_spec(D), _row_spec(D), _row_spec(1),
                  _full_spec((1, D))],
        out_specs=_row_spec(D),
        out_shape=jax.ShapeDtypeStruct((npad, D), jnp.float32),
    )


@functools.cache
def _make_tc_pool():
    return pl.pallas_call(
        _tc_pool_body,
        in_specs=[pl.BlockSpec((NW, GP, D), lambda: (0, 0, 0)),
                  pl.BlockSpec((D, 1), lambda: (0, 0)),
                  pl.BlockSpec((1, 1), lambda: (0, 0), memory_space=pltpu.SMEM)],
        out_specs=pl.BlockSpec((GP, 1), lambda: (0, 0)),
        out_shape=jax.ShapeDtypeStruct((GP, 1), jnp.float32),
    )


def kernel(x, edge_index, batch, emb, W1, b1, W2, b2, W3, b3, Wf, bf):
    n = x.shape[0]
    e = edge_index.shape[1]
    npad = ((n + NW * 16 - 1) // (NW * 16)) * (NW * 16)
    estep = NW * EK * NB
    ngrp = (e + estep - 1) // estep
    epad = ngrp * estep

    x_p = jnp.concatenate([x.astype(jnp.int32),
                           jnp.zeros((npad - n,), jnp.int32)])
    batch_p = jnp.concatenate([batch.astype(jnp.int32),
                               jnp.zeros((npad - n,), jnp.int32)])
    epad_fill = jnp.full((epad - e,), npad - 1, jnp.int32)
    src = jnp.concatenate([edge_index[0].astype(jnp.int32), epad_fill])
    dst = jnp.concatenate([edge_index[1].astype(jnp.int32), epad_fill])

    rps = npad // NS
    z128 = jnp.zeros((rps, D), jnp.float32)
    zpool = jnp.zeros((GP * D,), jnp.float32)
    ones128 = jnp.ones((EK, D), jnp.float32)

    deg2, h0 = _make_prep(npad, ngrp)(x_p, dst, emb, z128, ones128)
    dinv, g1 = _make_tc1(npad, n)(deg2[0], deg2[1], h0, W1)

    b1r = b1.reshape(1, D)
    b2r = b2.reshape(1, D)
    b3r = b3.reshape(1, D)

    sh1 = _make_scatter(npad, ngrp)(g1, src, dst, z128)
    g2 = _make_tc_mid(npad, n)(sh1[0], sh1[1], g1, dinv, b1r, W2)
    sh2 = _make_scatter(npad, ngrp)(g2, src, dst, z128)
    g3 = _make_tc_mid(npad, n)(sh2[0], sh2[1], g2, dinv, b2r, W3)
    sh3 = _make_scatter(npad, ngrp)(g3, src, dst, z128)
    h3 = _make_tc_last(npad, n)(sh3[0], sh3[1], g3, dinv, b3r)

    pool = _make_pool(npad)(h3.reshape(-1), batch_p, zpool)
    out = _make_tc_pool()(pool.reshape(NW, GP, D), Wf, bf.reshape(1, 1))
    return out[:G, 0]


# speedup vs baseline: 2.3894x; 2.3894x over previous
import functools

import jax
import jax.numpy as jnp
from jax import lax
from jax.experimental import pallas as pl
from jax.experimental.pallas import tpu as pltpu
from jax.experimental.pallas import tpu_sc as plsc

NC = 2
NS = 16
NW = NC * NS
D = 128
G = 100
GP = 104
EK = 64
NB = 4


@functools.cache
def _mesh():
    return plsc.VectorSubcoreMesh(core_axis_name="c", subcore_axis_name="s",
                                  num_cores=NC, num_subcores=NS)


def _prep_body(npad, ngrp, x_hbm, dst_hbm, emb_hbm, z128_hbm, ones_hbm,
               deg_hbm, h0_hbm, idxv, rows_v, ones_v, acc, isem, gsem, ssem):
    c = lax.axis_index("c")
    s = lax.axis_index("s")
    w = c * NS + s
    rpw = npad // NW
    rps = npad // NS

    nch = rpw // EK
    for start in range(0, nch, NB):
        bs = range(min(NB, nch - start))
        dl = [pltpu.async_copy(
            x_hbm.at[pl.ds(w * rpw + (start + b) * EK, EK)],
            idxv.at[b], isem) for b in bs]
        for dd in dl:
            dd.wait()
        dl = [pltpu.async_copy(emb_hbm.at[idxv.at[b]], rows_v.at[b], gsem)
              for b in bs]
        for dd in dl:
            dd.wait()
        dl = [pltpu.async_copy(
            rows_v.at[b], h0_hbm.at[pl.ds(w * rpw + (start + b) * EK, EK)],
            ssem) for b in bs]
        for dd in dl:
            dd.wait()

    pltpu.sync_copy(z128_hbm, acc.at[pl.ds(s * rps, rps)])
    pltpu.sync_copy(ones_hbm, ones_v)
    plsc.subcore_barrier()

    epw = ngrp * NB * EK

    def grp(gi, _):
        ds1 = [pltpu.async_copy(
            dst_hbm.at[pl.ds(w * epw + (gi * NB + b) * EK, EK)],
            idxv.at[b], isem) for b in range(NB)]
        for dd in ds1:
            dd.wait()
        ds2 = [pltpu.async_copy(ones_v, acc.at[idxv.at[b]], ssem, add=True)
               for b in range(NB)]
        for dd in ds2:
            dd.wait()
        return 0

    lax.fori_loop(0, ngrp, grp, 0)
    plsc.subcore_barrier()
    pltpu.sync_copy(acc.at[pl.ds(s * rps, rps)],
                    deg_hbm.at[c, pl.ds(s * rps, rps)])


def _scatter_body(npad, ngrp, g_hbm, src_hbm, dst_hbm, z128_hbm,
                  sh_hbm, srcv, dstv, rows_v, acc, isem, gsem, ssem):
    c = lax.axis_index("c")
    s = lax.axis_index("s")
    w = c * NS + s
    rps = npad // NS

    pltpu.sync_copy(z128_hbm, acc.at[pl.ds(s * rps, rps)])
    plsc.subcore_barrier()

    epw = ngrp * NB * EK

    def grp(gi, _):
        ds1 = []
        for b in range(NB):
            base = w * epw + (gi * NB + b) * EK
            ds1.append(pltpu.async_copy(src_hbm.at[pl.ds(base, EK)],
                                        srcv.at[b], isem))
            ds1.append(pltpu.async_copy(dst_hbm.at[pl.ds(base, EK)],
                                        dstv.at[b], isem))
        for dd in ds1:
            dd.wait()
        ds2 = [pltpu.async_copy(g_hbm.at[srcv.at[b]], rows_v.at[b], gsem)
               for b in range(NB)]
        for dd in ds2:
            dd.wait()
        ds3 = [pltpu.async_copy(rows_v.at[b], acc.at[dstv.at[b]], ssem,
                                add=True) for b in range(NB)]
        for dd in ds3:
            dd.wait()
        return 0

    lax.fori_loop(0, ngrp, grp, 0)
    plsc.subcore_barrier()
    pltpu.sync_copy(acc.at[pl.ds(s * rps, rps)],
                    sh_hbm.at[c, pl.ds(s * rps, rps)])


def _pool_body(npad, h3_hbm, batch_hbm, zpool_hbm, pool_hbm,
               buf, rows_v, bv):
    c = lax.axis_index("c")
    s = lax.axis_index("s")
    w = c * NS + s
    rpw = npad // NW

    pltpu.sync_copy(zpool_hbm, buf)
    base = w * rpw
    pltpu.sync_copy(h3_hbm.at[pl.ds(base * D, rpw * D)], rows_v)
    pltpu.sync_copy(batch_hbm.at[pl.ds(base, rpw)], bv)

    def row_blk(jb, _):
        bvec = bv[pl.ds(jb * 16, 16)]
        for ii in range(16):
            gb = bvec[ii] * D
            rb = (jb * 16 + ii) * D
            for k in range(D // 16):
                v = rows_v[pl.ds(rb + k * 16, 16)]
                cur = buf[pl.ds(gb + k * 16, 16)]
                buf[pl.ds(gb + k * 16, 16)] = jnp.maximum(cur, v)
        return 0

    lax.fori_loop(0, rpw // 16, row_blk, 0)
    pltpu.sync_copy(buf, pool_hbm.at[w])


@functools.cache
def _make_prep(npad, ngrp):
    return pl.kernel(
        functools.partial(_prep_body, npad, ngrp),
        out_type=(jax.ShapeDtypeStruct((NC, npad, D), jnp.float32),
                  jax.ShapeDtypeStruct((npad, D), jnp.float32)),
        mesh=_mesh(),
        scratch_types=[
            pltpu.VMEM((NB, EK), jnp.int32),
            pltpu.VMEM((NB, EK, D), jnp.float32),
            pltpu.VMEM((EK, D), jnp.float32),
            pltpu.VMEM_SHARED((npad, D), jnp.float32),
            pltpu.SemaphoreType.DMA,
            pltpu.SemaphoreType.DMA,
            pltpu.SemaphoreType.DMA,
        ],
    )


@functools.cache
def _make_scatter(npad, ngrp):
    return pl.kernel(
        functools.partial(_scatter_body, npad, ngrp),
        out_type=jax.ShapeDtypeStruct((NC, npad, D), jnp.float32),
        mesh=_mesh(),
        scratch_types=[
            pltpu.VMEM((NB, EK), jnp.int32),
            pltpu.VMEM((NB, EK), jnp.int32),
            pltpu.VMEM((NB, EK, D), jnp.float32),
            pltpu.VMEM_SHARED((npad, D), jnp.float32),
            pltpu.SemaphoreType.DMA,
            pltpu.SemaphoreType.DMA,
            pltpu.SemaphoreType.DMA,
        ],
    )


@functools.cache
def _make_pool(npad):
    rpw = npad // NW
    return pl.kernel(
        functools.partial(_pool_body, npad),
        out_type=jax.ShapeDtypeStruct((NW, GP * D), jnp.float32),
        mesh=_mesh(),
        scratch_types=[
            pltpu.VMEM((GP * D,), jnp.float32),
            pltpu.VMEM((rpw * D,), jnp.float32),
            pltpu.VMEM((rpw,), jnp.int32),
        ],
    )


BLK = 256


def _rowmask(nreal, val):
    ridx = pl.program_id(0) * BLK + lax.broadcasted_iota(jnp.int32, val.shape, 0)
    return jnp.where(ridx < nreal, val, 0.0)


def _tc1_body(nreal, dega, degb, h0, W, dinv_o, g_o):
    dinv = lax.rsqrt(dega[:, :1] + degb[:, :1] + 1.0)
    dinv_o[...] = dinv
    g = dinv * jnp.dot(h0[...], W[...], preferred_element_type=jnp.float32)
    g_o[...] = _rowmask(nreal, g)


def _tc_mid_body(nreal, sa, sb, g, dinv, b, W, out):
    dv = dinv[...]
    h = jnp.maximum(dv * (sa[...] + sb[...] + g[...]) + b[...], 0.0)
    out[...] = _rowmask(nreal, dv * jnp.dot(h, W[...],
                                            preferred_element_type=jnp.float32))


def _tc_last_body(nreal, sa, sb, g, dinv, b, out):
    h = jnp.maximum(dinv[...] * (sa[...] + sb[...] + g[...]) + b[...], 0.0)
    out[...] = _rowmask(nreal, h)


def _tc_pool_body(pool, Wf, bf, out):
    pooled = jnp.max(pool[...], axis=0)
    out[...] = jnp.dot(pooled, Wf[...], preferred_element_type=jnp.float32) + bf[0, 0]


def _row_spec(width):
    return pl.BlockSpec((BLK, width), lambda i: (i, 0))


def _full_spec(shape):
    return pl.BlockSpec(shape, lambda i: tuple(0 for _ in shape))


@functools.cache
def _make_tc1(npad, nreal):
    return pl.pallas_call(
        functools.partial(_tc1_body, nreal),
        grid=(npad // BLK,),
        in_specs=[_row_spec(D), _row_spec(D), _row_spec(D), _full_spec((D, D))],
        out_specs=[_row_spec(1), _row_spec(D)],
        out_shape=(jax.ShapeDtypeStruct((npad, 1), jnp.float32),
                   jax.ShapeDtypeStruct((npad, D), jnp.float32)),
    )


@functools.cache
def _make_tc_mid(npad, nreal):
    return pl.pallas_call(
        functools.partial(_tc_mid_body, nreal),
        grid=(npad // BLK,),
        in_specs=[_row_spec(D), _row_spec(D), _row_spec(D), _row_spec(1),
                  _full_spec((1, D)), _full_spec((D, D))],
        out_specs=_row_spec(D),
        out_shape=jax.ShapeDtypeStruct((npad, D), jnp.float32),
    )


@functools.cache
def _make_tc_last(npad, nreal):
    return pl.pallas_call(
        functools.partial(_tc_last_body, nreal),
        grid=(npad // BLK,),
        in_specs=[_row_spec(D), _row_spec(D), _row_spec(D), _row_spec(1),
                  _full_spec((1, D))],
        out_specs=_row_spec(D),
        out_shape=jax.ShapeDtypeStruct((npad, D), jnp.float32),
    )


@functools.cache
def _make_tc_pool():
    return pl.pallas_call(
        _tc_pool_body,
        in_specs=[pl.BlockSpec((NW, GP, D), lambda: (0, 0, 0)),
                  pl.BlockSpec((D, 1), lambda: (0, 0)),
                  pl.BlockSpec((1, 1), lambda: (0, 0), memory_space=pltpu.SMEM)],
        out_specs=pl.BlockSpec((GP, 1), lambda: (0, 0)),
        out_shape=jax.ShapeDtypeStruct((GP, 1), jnp.float32),
    )


def kernel(x, edge_index, batch, emb, W1, b1, W2, b2, W3, b3, Wf, bf):
    n = x.shape[0]
    e = edge_index.shape[1]
    npad = ((n + NW * 16 - 1) // (NW * 16)) * (NW * 16)
    estep = NW * EK * NB
    ngrp = (e + estep - 1) // estep
    epad = ngrp * estep

    x_p = jnp.concatenate([x.astype(jnp.int32),
                           jnp.zeros((npad - n,), jnp.int32)])
    batch_p = jnp.concatenate([batch.astype(jnp.int32),
                               jnp.zeros((npad - n,), jnp.int32)])
    epad_fill = n + jnp.arange(epad - e, dtype=jnp.int32) % (npad - n)
    src = jnp.concatenate([edge_index[0].astype(jnp.int32), epad_fill])
    dst = jnp.concatenate([edge_index[1].astype(jnp.int32), epad_fill])

    rps = npad // NS
    z128 = jnp.zeros((rps, D), jnp.float32)
    zpool = jnp.zeros((GP * D,), jnp.float32)
    ones128 = jnp.ones((EK, D), jnp.float32)

    deg2, h0 = _make_prep(npad, ngrp)(x_p, dst, emb, z128, ones128)
    dinv, g1 = _make_tc1(npad, n)(deg2[0], deg2[1], h0, W1)

    b1r = b1.reshape(1, D)
    b2r = b2.reshape(1, D)
    b3r = b3.reshape(1, D)

    sh1 = _make_scatter(npad, ngrp)(g1, src, dst, z128)
    g2 = _make_tc_mid(npad, n)(sh1[0], sh1[1], g1, dinv, b1r, W2)
    sh2 = _make_scatter(npad, ngrp)(g2, src, dst, z128)
    g3 = _make_tc_mid(npad, n)(sh2[0], sh2[1], g2, dinv, b2r, W3)
    sh3 = _make_scatter(npad, ngrp)(g3, src, dst, z128)
    h3 = _make_tc_last(npad, n)(sh3[0], sh3[1], g3, dinv, b3r)

    pool = _make_pool(npad)(h3.reshape(-1), batch_p, zpool)
    out = _make_tc_pool()(pool.reshape(NW, GP, D), Wf, bf.reshape(1, 1))
    return out[:G, 0]


# speedup vs baseline: 3.0623x; 1.2816x over previous
import functools

import jax
import jax.numpy as jnp
from jax import lax
from jax.experimental import pallas as pl
from jax.experimental.pallas import tpu as pltpu
from jax.experimental.pallas import tpu_sc as plsc

NC = 2
NS = 16
NW = NC * NS
D = 128
G = 100
GP = 104
EK = 64
NB = 4


@functools.cache
def _mesh():
    return plsc.VectorSubcoreMesh(core_axis_name="c", subcore_axis_name="s",
                                  num_cores=NC, num_subcores=NS)


def _prep_body(npad, ngrp, x_hbm, dst_hbm, emb_hbm, z128_hbm, ones_hbm,
               deg_hbm, h0_hbm, idxv, rows_v, ones_v, acc, *sems):
    ig = sems[:2 * NB]
    ss = sems[2 * NB:3 * NB]
    hsem = sems[3 * NB]
    c = lax.axis_index("c")
    s = lax.axis_index("s")
    w = c * NS + s
    rpw = npad // NW
    rps = npad // NS

    nch = rpw // EK
    for start in range(0, nch, NB):
        bs = range(min(NB, nch - start))
        dl = [pltpu.async_copy(
            x_hbm.at[pl.ds(w * rpw + (start + b) * EK, EK)],
            idxv.at[b], hsem) for b in bs]
        for dd in dl:
            dd.wait()
        dl = [pltpu.async_copy(emb_hbm.at[idxv.at[b]], rows_v.at[b], hsem)
              for b in bs]
        for dd in dl:
            dd.wait()
        dl = [pltpu.async_copy(
            rows_v.at[b], h0_hbm.at[pl.ds(w * rpw + (start + b) * EK, EK)],
            hsem) for b in bs]
        for dd in dl:
            dd.wait()

    pltpu.sync_copy(z128_hbm, acc.at[pl.ds(s * rps, rps)])
    pltpu.sync_copy(ones_hbm, ones_v)
    plsc.subcore_barrier()

    epw = ngrp * NB * EK

    def fire_idx(gidx, sb, b):
        base = w * epw + (gidx * NB + b) * EK
        pltpu.async_copy(dst_hbm.at[pl.ds(base, EK)], idxv.at[sb], ig[sb])

    for b in range(NB):
        fire_idx(0, b, b)

    def g2_body(g2, _):
        for half in range(2):
            gidx = 2 * g2 + half
            q = half * NB
            for b in range(NB):
                sb = q + b
                ob = (NB - q) + b

                @pl.when(gidx > 0)
                def _():
                    pltpu.make_async_copy(ones_hbm, ones_v, ss[b]).wait()

                @pl.when(gidx + 1 < ngrp)
                def _():
                    fire_idx(gidx + 1, ob, b)

                pltpu.make_async_copy(dst_hbm.at[pl.ds(0, EK)],
                                      idxv.at[sb], ig[sb]).wait()
                pltpu.async_copy(ones_v, acc.at[idxv.at[sb]], ss[b], add=True)
        return 0

    lax.fori_loop(0, ngrp // 2, g2_body, 0)
    for b in range(NB):
        pltpu.make_async_copy(ones_hbm, ones_v, ss[b]).wait()
    plsc.subcore_barrier()
    pltpu.sync_copy(acc.at[pl.ds(s * rps, rps)],
                    deg_hbm.at[c, pl.ds(s * rps, rps)])


def _scatter_body(npad, ngrp, g_hbm, src_hbm, dst_hbm, z128_hbm,
                  sh_hbm, srcv, dstv, rows_v, acc, *sems):
    ig = sems[:2 * NB]
    gs = sems[2 * NB:3 * NB]
    ss = sems[3 * NB:4 * NB]
    c = lax.axis_index("c")
    s = lax.axis_index("s")
    w = c * NS + s
    rps = npad // NS

    pltpu.sync_copy(z128_hbm, acc.at[pl.ds(s * rps, rps)])
    plsc.subcore_barrier()

    epw = ngrp * NB * EK

    def fire_idx(gidx, sb, b):
        base = w * epw + (gidx * NB + b) * EK
        pltpu.async_copy(src_hbm.at[pl.ds(base, EK)], srcv.at[sb], ig[sb])
        pltpu.async_copy(dst_hbm.at[pl.ds(base, EK)], dstv.at[sb], ig[sb])

    for b in range(NB):
        fire_idx(0, b, b)

    def g2_body(g2, _):
        for half in range(2):
            gidx = 2 * g2 + half
            q = half * NB
            for b in range(NB):
                sb = q + b
                ob = (NB - q) + b

                @pl.when(gidx > 0)
                def _():
                    pltpu.make_async_copy(g_hbm.at[pl.ds(0, EK)],
                                          rows_v.at[b], ss[b]).wait()

                @pl.when(gidx + 1 < ngrp)
                def _():
                    fire_idx(gidx + 1, ob, b)

                pltpu.make_async_copy(src_hbm.at[pl.ds(0, EK)],
                                      srcv.at[sb], ig[sb]).wait()
                pltpu.make_async_copy(src_hbm.at[pl.ds(0, EK)],
                                      dstv.at[sb], ig[sb]).wait()
                pltpu.async_copy(g_hbm.at[srcv.at[sb]], rows_v.at[b], gs[b])
            for b in range(NB):
                sb = q + b
                pltpu.make_async_copy(g_hbm.at[pl.ds(0, EK)],
                                      rows_v.at[b], gs[b]).wait()
                pltpu.async_copy(rows_v.at[b], acc.at[dstv.at[sb]], ss[b],
                                 add=True)
        return 0

    lax.fori_loop(0, ngrp // 2, g2_body, 0)
    for b in range(NB):
        pltpu.make_async_copy(g_hbm.at[pl.ds(0, EK)], rows_v.at[b],
                              ss[b]).wait()
    plsc.subcore_barrier()
    pltpu.sync_copy(acc.at[pl.ds(s * rps, rps)],
                    sh_hbm.at[c, pl.ds(s * rps, rps)])


def _pool_body(npad, h3_hbm, batch_hbm, zpool_hbm, pool_hbm,
               buf, rows_v, bv):
    c = lax.axis_index("c")
    s = lax.axis_index("s")
    w = c * NS + s
    rpw = npad // NW

    pltpu.sync_copy(zpool_hbm, buf)
    base = w * rpw
    pltpu.sync_copy(h3_hbm.at[pl.ds(base * D, rpw * D)], rows_v)
    pltpu.sync_copy(batch_hbm.at[pl.ds(base, rpw)], bv)

    def row_blk(jb, _):
        bvec = bv[pl.ds(jb * 16, 16)]
        for ii in range(16):
            gb = bvec[ii] * D
            rb = (jb * 16 + ii) * D
            for k in range(D // 16):
                v = rows_v[pl.ds(rb + k * 16, 16)]
                cur = buf[pl.ds(gb + k * 16, 16)]
                buf[pl.ds(gb + k * 16, 16)] = jnp.maximum(cur, v)
        return 0

    lax.fori_loop(0, rpw // 16, row_blk, 0)
    pltpu.sync_copy(buf, pool_hbm.at[w])


@functools.cache
def _make_prep(npad, ngrp):
    return pl.kernel(
        functools.partial(_prep_body, npad, ngrp),
        out_type=(jax.ShapeDtypeStruct((NC, npad, D), jnp.float32),
                  jax.ShapeDtypeStruct((npad, D), jnp.float32)),
        mesh=_mesh(),
        scratch_types=[
            pltpu.VMEM((2 * NB, EK), jnp.int32),
            pltpu.VMEM((NB, EK, D), jnp.float32),
            pltpu.VMEM((EK, D), jnp.float32),
            pltpu.VMEM_SHARED((npad, D), jnp.float32),
        ] + [pltpu.SemaphoreType.DMA] * (3 * NB + 1),
    )


@functools.cache
def _make_scatter(npad, ngrp):
    return pl.kernel(
        functools.partial(_scatter_body, npad, ngrp),
        out_type=jax.ShapeDtypeStruct((NC, npad, D), jnp.float32),
        mesh=_mesh(),
        scratch_types=[
            pltpu.VMEM((2 * NB, EK), jnp.int32),
            pltpu.VMEM((2 * NB, EK), jnp.int32),
            pltpu.VMEM((NB, EK, D), jnp.float32),
            pltpu.VMEM_SHARED((npad, D), jnp.float32),
        ] + [pltpu.SemaphoreType.DMA] * (4 * NB),
    )


@functools.cache
def _make_pool(npad):
    rpw = npad // NW
    return pl.kernel(
        functools.partial(_pool_body, npad),
        out_type=jax.ShapeDtypeStruct((NW, GP * D), jnp.float32),
        mesh=_mesh(),
        scratch_types=[
            pltpu.VMEM((GP * D,), jnp.float32),
            pltpu.VMEM((rpw * D,), jnp.float32),
            pltpu.VMEM((rpw,), jnp.int32),
        ],
    )


BLK = 256


def _rowmask(nreal, val):
    ridx = pl.program_id(0) * BLK + lax.broadcasted_iota(jnp.int32, val.shape, 0)
    return jnp.where(ridx < nreal, val, 0.0)


def _tc1_body(nreal, dega, degb, h0, W, dinv_o, g_o):
    dinv = lax.rsqrt(dega[:, :1] + degb[:, :1] + 1.0)
    dinv_o[...] = dinv
    g = dinv * jnp.dot(h0[...], W[...], preferred_element_type=jnp.float32)
    g_o[...] = _rowmask(nreal, g)


def _tc_mid_body(nreal, sa, sb, g, dinv, b, W, out):
    dv = dinv[...]
    h = jnp.maximum(dv * (sa[...] + sb[...] + g[...]) + b[...], 0.0)
    out[...] = _rowmask(nreal, dv * jnp.dot(h, W[...],
                                            preferred_element_type=jnp.float32))


def _tc_last_body(nreal, sa, sb, g, dinv, b, out):
    h = jnp.maximum(dinv[...] * (sa[...] + sb[...] + g[...]) + b[...], 0.0)
    out[...] = _rowmask(nreal, h)


def _tc_pool_body(pool, Wf, bf, out):
    pooled = jnp.max(pool[...], axis=0)
    out[...] = jnp.dot(pooled, Wf[...], preferred_element_type=jnp.float32) + bf[0, 0]


def _row_spec(width):
    return pl.BlockSpec((BLK, width), lambda i: (i, 0))


def _full_spec(shape):
    return pl.BlockSpec(shape, lambda i: tuple(0 for _ in shape))


@functools.cache
def _make_tc1(npad, nreal):
    return pl.pallas_call(
        functools.partial(_tc1_body, nreal),
        grid=(npad // BLK,),
        in_specs=[_row_spec(D), _row_spec(D), _row_spec(D), _full_spec((D, D))],
        out_specs=[_row_spec(1), _row_spec(D)],
        out_shape=(jax.ShapeDtypeStruct((npad, 1), jnp.float32),
                   jax.ShapeDtypeStruct((npad, D), jnp.float32)),
    )


@functools.cache
def _make_tc_mid(npad, nreal):
    return pl.pallas_call(
        functools.partial(_tc_mid_body, nreal),
        grid=(npad // BLK,),
        in_specs=[_row_spec(D), _row_spec(D), _row_spec(D), _row_spec(1),
                  _full_spec((1, D)), _full_spec((D, D))],
        out_specs=_row_spec(D),
        out_shape=jax.ShapeDtypeStruct((npad, D), jnp.float32),
    )


@functools.cache
def _make_tc_last(npad, nreal):
    return pl.pallas_call(
        functools.partial(_tc_last_body, nreal),
        grid=(npad // BLK,),
        in_specs=[_row_spec(D), _row_spec(D), _row_spec(D), _row_spec(1),
                  _full_spec((1, D))],
        out_specs=_row_spec(D),
        out_shape=jax.ShapeDtypeStruct((npad, D), jnp.float32),
    )


@functools.cache
def _make_tc_pool():
    return pl.pallas_call(
        _tc_pool_body,
        in_specs=[pl.BlockSpec((NW, GP, D), lambda: (0, 0, 0)),
                  pl.BlockSpec((D, 1), lambda: (0, 0)),
                  pl.BlockSpec((1, 1), lambda: (0, 0), memory_space=pltpu.SMEM)],
        out_specs=pl.BlockSpec((GP, 1), lambda: (0, 0)),
        out_shape=jax.ShapeDtypeStruct((GP, 1), jnp.float32),
    )


def kernel(x, edge_index, batch, emb, W1, b1, W2, b2, W3, b3, Wf, bf):
    n = x.shape[0]
    e = edge_index.shape[1]
    npad = ((n + NW * 16 - 1) // (NW * 16)) * (NW * 16)
    estep = NW * EK * NB
    ngrp = (e + estep - 1) // estep
    ngrp += ngrp % 2
    epad = ngrp * estep

    x_p = jnp.concatenate([x.astype(jnp.int32),
                           jnp.zeros((npad - n,), jnp.int32)])
    batch_p = jnp.concatenate([batch.astype(jnp.int32),
                               jnp.zeros((npad - n,), jnp.int32)])
    epad_fill = n + jnp.arange(epad - e, dtype=jnp.int32) % (npad - n)
    src = jnp.concatenate([edge_index[0].astype(jnp.int32), epad_fill])
    dst = jnp.concatenate([edge_index[1].astype(jnp.int32), epad_fill])

    rps = npad // NS
    z128 = jnp.zeros((rps, D), jnp.float32)
    zpool = jnp.zeros((GP * D,), jnp.float32)
    ones128 = jnp.ones((EK, D), jnp.float32)

    deg2, h0 = _make_prep(npad, ngrp)(x_p, dst, emb, z128, ones128)
    dinv, g1 = _make_tc1(npad, n)(deg2[0], deg2[1], h0, W1)

    b1r = b1.reshape(1, D)
    b2r = b2.reshape(1, D)
    b3r = b3.reshape(1, D)

    sh1 = _make_scatter(npad, ngrp)(g1, src, dst, z128)
    g2 = _make_tc_mid(npad, n)(sh1[0], sh1[1], g1, dinv, b1r, W2)
    sh2 = _make_scatter(npad, ngrp)(g2, src, dst, z128)
    g3 = _make_tc_mid(npad, n)(sh2[0], sh2[1], g2, dinv, b2r, W3)
    sh3 = _make_scatter(npad, ngrp)(g3, src, dst, z128)
    h3 = _make_tc_last(npad, n)(sh3[0], sh3[1], g3, dinv, b3r)

    pool = _make_pool(npad)(h3.reshape(-1), batch_p, zpool)
    out = _make_tc_pool()(pool.reshape(NW, GP, D), Wf, bf.reshape(1, 1))
    return out[:G, 0]


# speedup vs baseline: 3.1162x; 1.0176x over previous
import functools

import jax
import jax.numpy as jnp
from jax import lax
from jax.experimental import pallas as pl
from jax.experimental.pallas import tpu as pltpu
from jax.experimental.pallas import tpu_sc as plsc

NC = 2
NS = 16
NW = NC * NS
D = 128
G = 100
GP = 104
EK = 64
NB = 5


@functools.cache
def _mesh():
    return plsc.VectorSubcoreMesh(core_axis_name="c", subcore_axis_name="s",
                                  num_cores=NC, num_subcores=NS)


def _prep_body(npad, ngrp, x_hbm, dst_hbm, emb_hbm, z128_hbm, ones_hbm,
               deg_hbm, h0_hbm, idxv, rows_v, ones_v, acc, *sems):
    ig = sems[:2 * NB]
    ss = sems[2 * NB:3 * NB]
    hsem = sems[3 * NB]
    c = lax.axis_index("c")
    s = lax.axis_index("s")
    w = c * NS + s
    rpw = npad // NW
    rps = npad // NS

    nbh = 4
    nch = rpw // EK
    for start in range(0, nch, nbh):
        bs = range(min(nbh, nch - start))
        dl = [pltpu.async_copy(
            x_hbm.at[pl.ds(w * rpw + (start + b) * EK, EK)],
            idxv.at[b], hsem) for b in bs]
        for dd in dl:
            dd.wait()
        dl = [pltpu.async_copy(emb_hbm.at[idxv.at[b]], rows_v.at[b], hsem)
              for b in bs]
        for dd in dl:
            dd.wait()
        dl = [pltpu.async_copy(
            rows_v.at[b], h0_hbm.at[pl.ds(w * rpw + (start + b) * EK, EK)],
            hsem) for b in bs]
        for dd in dl:
            dd.wait()

    pltpu.sync_copy(z128_hbm, acc.at[pl.ds(s * rps, rps)])
    pltpu.sync_copy(ones_hbm, ones_v)
    plsc.subcore_barrier()

    epw = ngrp * NB * EK

    def fire_idx(gidx, sb, b):
        base = w * epw + (gidx * NB + b) * EK
        pltpu.async_copy(dst_hbm.at[pl.ds(base, EK)], idxv.at[sb], ig[sb])

    for b in range(NB):
        fire_idx(0, b, b)

    def g2_body(g2, _):
        for half in range(2):
            gidx = 2 * g2 + half
            q = half * NB
            for b in range(NB):
                sb = q + b
                ob = (NB - q) + b

                @pl.when(gidx > 0)
                def _():
                    pltpu.make_async_copy(ones_hbm, ones_v, ss[b]).wait()

                @pl.when(gidx + 1 < ngrp)
                def _():
                    fire_idx(gidx + 1, ob, b)

                pltpu.make_async_copy(dst_hbm.at[pl.ds(0, EK)],
                                      idxv.at[sb], ig[sb]).wait()
                pltpu.async_copy(ones_v, acc.at[idxv.at[sb]], ss[b], add=True)
        return 0

    lax.fori_loop(0, ngrp // 2, g2_body, 0)
    for b in range(NB):
        pltpu.make_async_copy(ones_hbm, ones_v, ss[b]).wait()
    plsc.subcore_barrier()
    pltpu.sync_copy(acc.at[pl.ds(s * rps, rps)],
                    deg_hbm.at[c, pl.ds(s * rps, rps)])


def _scatter_body(npad, ngrp, g_hbm, src_hbm, dst_hbm, z128_hbm,
                  sh_hbm, srcv, dstv, rows_v, acc, *sems):
    ig = sems[:2 * NB]
    gs = sems[2 * NB:3 * NB]
    ss = sems[3 * NB:4 * NB]
    c = lax.axis_index("c")
    s = lax.axis_index("s")
    w = c * NS + s
    rps = npad // NS

    pltpu.sync_copy(z128_hbm, acc.at[pl.ds(s * rps, rps)])
    plsc.subcore_barrier()

    epw = ngrp * NB * EK

    def fire_idx(gidx, sb, b):
        base = w * epw + (gidx * NB + b) * EK
        pltpu.async_copy(src_hbm.at[pl.ds(base, EK)], srcv.at[sb], ig[sb])
        pltpu.async_copy(dst_hbm.at[pl.ds(base, EK)], dstv.at[sb], ig[sb])

    for b in range(NB):
        fire_idx(0, b, b)

    def g2_body(g2, _):
        for half in range(2):
            gidx = 2 * g2 + half
            q = half * NB
            for b in range(NB):
                sb = q + b
                ob = (NB - q) + b

                @pl.when(gidx > 0)
                def _():
                    pltpu.make_async_copy(g_hbm.at[pl.ds(0, EK)],
                                          rows_v.at[b], ss[b]).wait()

                @pl.when(gidx + 1 < ngrp)
                def _():
                    fire_idx(gidx + 1, ob, b)

                pltpu.make_async_copy(src_hbm.at[pl.ds(0, EK)],
                                      srcv.at[sb], ig[sb]).wait()
                pltpu.make_async_copy(src_hbm.at[pl.ds(0, EK)],
                                      dstv.at[sb], ig[sb]).wait()
                pltpu.async_copy(g_hbm.at[srcv.at[sb]], rows_v.at[b], gs[b])
            for b in range(NB):
                sb = q + b
                pltpu.make_async_copy(g_hbm.at[pl.ds(0, EK)],
                                      rows_v.at[b], gs[b]).wait()
                pltpu.async_copy(rows_v.at[b], acc.at[dstv.at[sb]], ss[b],
                                 add=True)
        return 0

    lax.fori_loop(0, ngrp // 2, g2_body, 0)
    for b in range(NB):
        pltpu.make_async_copy(g_hbm.at[pl.ds(0, EK)], rows_v.at[b],
                              ss[b]).wait()
    plsc.subcore_barrier()
    pltpu.sync_copy(acc.at[pl.ds(s * rps, rps)],
                    sh_hbm.at[c, pl.ds(s * rps, rps)])


def _pool_body(npad, h3_hbm, batch_hbm, zpool_hbm, pool_hbm,
               buf, rows_v, bv):
    c = lax.axis_index("c")
    s = lax.axis_index("s")
    w = c * NS + s
    rpw = npad // NW

    pltpu.sync_copy(zpool_hbm, buf)
    base = w * rpw
    pltpu.sync_copy(h3_hbm.at[pl.ds(base * D, rpw * D)], rows_v)
    pltpu.sync_copy(batch_hbm.at[pl.ds(base, rpw)], bv)

    def row_blk(jb, _):
        bvec = bv[pl.ds(jb * 16, 16)]
        for ii in range(16):
            gb = bvec[ii] * D
            rb = (jb * 16 + ii) * D
            for k in range(D // 16):
                v = rows_v[pl.ds(rb + k * 16, 16)]
                cur = buf[pl.ds(gb + k * 16, 16)]
                buf[pl.ds(gb + k * 16, 16)] = jnp.maximum(cur, v)
        return 0

    lax.fori_loop(0, rpw // 16, row_blk, 0)
    pltpu.sync_copy(buf, pool_hbm.at[w])


@functools.cache
def _make_prep(npad, ngrp):
    return pl.kernel(
        functools.partial(_prep_body, npad, ngrp),
        out_type=(jax.ShapeDtypeStruct((NC, npad, D), jnp.float32),
                  jax.ShapeDtypeStruct((npad, D), jnp.float32)),
        mesh=_mesh(),
        scratch_types=[
            pltpu.VMEM((2 * NB, EK), jnp.int32),
            pltpu.VMEM((4, EK, D), jnp.float32),
            pltpu.VMEM((EK, D), jnp.float32),
            pltpu.VMEM_SHARED((npad, D), jnp.float32),
        ] + [pltpu.SemaphoreType.DMA] * (3 * NB + 1),
    )


@functools.cache
def _make_scatter(npad, ngrp):
    return pl.kernel(
        functools.partial(_scatter_body, npad, ngrp),
        out_type=jax.ShapeDtypeStruct((NC, npad, D), jnp.float32),
        mesh=_mesh(),
        scratch_types=[
            pltpu.VMEM((2 * NB, EK), jnp.int32),
            pltpu.VMEM((2 * NB, EK), jnp.int32),
            pltpu.VMEM((NB, EK, D), jnp.float32),
            pltpu.VMEM_SHARED((npad, D), jnp.float32),
        ] + [pltpu.SemaphoreType.DMA] * (4 * NB),
    )


@functools.cache
def _make_pool(npad):
    rpw = npad // NW
    return pl.kernel(
        functools.partial(_pool_body, npad),
        out_type=jax.ShapeDtypeStruct((NW, GP * D), jnp.float32),
        mesh=_mesh(),
        scratch_types=[
            pltpu.VMEM((GP * D,), jnp.float32),
            pltpu.VMEM((rpw * D,), jnp.float32),
            pltpu.VMEM((rpw,), jnp.int32),
        ],
    )


BLK = 256


def _rowmask(nreal, val):
    ridx = pl.program_id(0) * BLK + lax.broadcasted_iota(jnp.int32, val.shape, 0)
    return jnp.where(ridx < nreal, val, 0.0)


def _tc1_body(nreal, dega, degb, h0, W, dinv_o, g_o):
    dinv = lax.rsqrt(dega[:, :1] + degb[:, :1] + 1.0)
    dinv_o[...] = dinv
    g = dinv * jnp.dot(h0[...], W[...], preferred_element_type=jnp.float32)
    g_o[...] = _rowmask(nreal, g)


def _tc_mid_body(nreal, sa, sb, g, dinv, b, W, out):
    dv = dinv[...]
    h = jnp.maximum(dv * (sa[...] + sb[...] + g[...]) + b[...], 0.0)
    out[...] = _rowmask(nreal, dv * jnp.dot(h, W[...],
                                            preferred_element_type=jnp.float32))


def _tc_last_body(nreal, sa, sb, g, dinv, b, out):
    h = jnp.maximum(dinv[...] * (sa[...] + sb[...] + g[...]) + b[...], 0.0)
    out[...] = _rowmask(nreal, h)


def _tc_pool_body(pool, Wf, bf, out):
    pooled = jnp.max(pool[...], axis=0)
    out[...] = jnp.dot(pooled, Wf[...], preferred_element_type=jnp.float32) + bf[0, 0]


def _row_spec(width):
    return pl.BlockSpec((BLK, width), lambda i: (i, 0))


def _full_spec(shape):
    return pl.BlockSpec(shape, lambda i: tuple(0 for _ in shape))


@functools.cache
def _make_tc1(npad, nreal):
    return pl.pallas_call(
        functools.partial(_tc1_body, nreal),
        grid=(npad // BLK,),
        in_specs=[_row_spec(D), _row_spec(D), _row_spec(D), _full_spec((D, D))],
        out_specs=[_row_spec(1), _row_spec(D)],
        out_shape=(jax.ShapeDtypeStruct((npad, 1), jnp.float32),
                   jax.ShapeDtypeStruct((npad, D), jnp.float32)),
    )


@functools.cache
def _make_tc_mid(npad, nreal):
    return pl.pallas_call(
        functools.partial(_tc_mid_body, nreal),
        grid=(npad // BLK,),
        in_specs=[_row_spec(D), _row_spec(D), _row_spec(D), _row_spec(1),
                  _full_spec((1, D)), _full_spec((D, D))],
        out_specs=_row_spec(D),
        out_shape=jax.ShapeDtypeStruct((npad, D), jnp.float32),
    )


@functools.cache
def _make_tc_last(npad, nreal):
    return pl.pallas_call(
        functools.partial(_tc_last_body, nreal),
        grid=(npad // BLK,),
        in_specs=[_row_spec(D), _row_spec(D), _row_spec(D), _row_spec(1),
                  _full_spec((1, D))],
        out_specs=_row_spec(D),
        out_shape=jax.ShapeDtypeStruct((npad, D), jnp.float32),
    )


@functools.cache
def _make_tc_pool():
    return pl.pallas_call(
        _tc_pool_body,
        in_specs=[pl.BlockSpec((NW, GP, D), lambda: (0, 0, 0)),
                  pl.BlockSpec((D, 1), lambda: (0, 0)),
                  pl.BlockSpec((1, 1), lambda: (0, 0), memory_space=pltpu.SMEM)],
        out_specs=pl.BlockSpec((GP, 1), lambda: (0, 0)),
        out_shape=jax.ShapeDtypeStruct((GP, 1), jnp.float32),
    )


def kernel(x, edge_index, batch, emb, W1, b1, W2, b2, W3, b3, Wf, bf):
    n = x.shape[0]
    e = edge_index.shape[1]
    npad = ((n + NW * 16 - 1) // (NW * 16)) * (NW * 16)
    estep = NW * EK * NB
    ngrp = (e + estep - 1) // estep
    ngrp += ngrp % 2
    epad = ngrp * estep

    x_p = jnp.concatenate([x.astype(jnp.int32),
                           jnp.zeros((npad - n,), jnp.int32)])
    batch_p = jnp.concatenate([batch.astype(jnp.int32),
                               jnp.zeros((npad - n,), jnp.int32)])
    epad_fill = n + jnp.arange(epad - e, dtype=jnp.int32) % (npad - n)
    src = jnp.concatenate([edge_index[0].astype(jnp.int32), epad_fill])
    dst = jnp.concatenate([edge_index[1].astype(jnp.int32), epad_fill])

    rps = npad // NS
    z128 = jnp.zeros((rps, D), jnp.float32)
    zpool = jnp.zeros((GP * D,), jnp.float32)
    ones128 = jnp.ones((EK, D), jnp.float32)

    deg2, h0 = _make_prep(npad, ngrp)(x_p, dst, emb, z128, ones128)
    dinv, g1 = _make_tc1(npad, n)(deg2[0], deg2[1], h0, W1)

    b1r = b1.reshape(1, D)
    b2r = b2.reshape(1, D)
    b3r = b3.reshape(1, D)

    sh1 = _make_scatter(npad, ngrp)(g1, src, dst, z128)
    g2 = _make_tc_mid(npad, n)(sh1[0], sh1[1], g1, dinv, b1r, W2)
    sh2 = _make_scatter(npad, ngrp)(g2, src, dst, z128)
    g3 = _make_tc_mid(npad, n)(sh2[0], sh2[1], g2, dinv, b2r, W3)
    sh3 = _make_scatter(npad, ngrp)(g3, src, dst, z128)
    h3 = _make_tc_last(npad, n)(sh3[0], sh3[1], g3, dinv, b3r)

    pool = _make_pool(npad)(h3.reshape(-1), batch_p, zpool)
    out = _make_tc_pool()(pool.reshape(NW, GP, D), Wf, bf.reshape(1, 1))
    return out[:G, 0]
